# Initial kernel scaffold; baseline (speedup 1.0000x reference)
import jax
import jax.numpy as jnp
from jax.experimental import pallas as pl


def kernel(x, fc1_w, fc1_b, fc2_w, fc2_b, gate_w, expert_w1, expert_b1, expert_w2, expert_b2, tower_w1, tower_b1, tower_w2, tower_b2):
    raise NotImplementedError("write your pallas kernel here")


# speedup vs baseline: 3.1131x; 3.1131x over previous
import functools

import jax
import jax.numpy as jnp
from jax import lax
from jax.experimental import pallas as pl
from jax.experimental.pallas import tpu as pltpu
from jax.experimental.pallas import tpu_sc as plsc

N, D, DIN, DFF, E, NT, TH = 4096, 768, 1536, 512, 64, 3, 128
CAP = 80
EC = 256
BUF_ROWS = E * EC
BN = 512
NB = N // BN
BT = 512
NBT = N // BT
NC, NS = 2, 16
NW = NC * NS
HP = lax.Precision.HIGHEST
F32 = jnp.float32


def _bottom_body(x_ref, w1_ref, b1_ref, w2_ref, b2_ref, gw_ref,
                 h_ref, dst_ref, scale_ref, imp_ref, cnt_ref):
    i = pl.program_id(0)

    @pl.when(i == 0)
    def _():
        cnt_ref[...] = jnp.zeros_like(cnt_ref)
        imp_ref[...] = jnp.zeros_like(imp_ref)

    xb = x_ref[...]
    h1 = jnp.maximum(
        jnp.dot(xb.astype(jnp.bfloat16), w1_ref[...].astype(jnp.bfloat16),
                preferred_element_type=F32) + b1_ref[...], 0.0)
    h = jnp.dot(h1.astype(jnp.bfloat16), w2_ref[...].astype(jnp.bfloat16),
                preferred_element_type=F32) + b2_ref[...]
    h_ref[...] = h
    lg = jnp.dot(h.astype(jnp.bfloat16), gw_ref[...].astype(jnp.bfloat16),
                 preferred_element_type=F32)

    iota_e = lax.broadcasted_iota(jnp.int32, (1, E), 1)
    r = lax.broadcasted_iota(jnp.int32, (BN, BN), 0)
    c = lax.broadcasted_iota(jnp.int32, (BN, BN), 1)
    ltri = (r > c).astype(F32)

    dcols, scols, icols = [], [], []
    for t in range(NT):
        l = lg[:, t * E:(t + 1) * E]
        m = jnp.max(l, axis=1, keepdims=True)
        ex = jnp.exp(l - m)
        s = jnp.sum(ex, axis=1, keepdims=True)
        gate = 1.0 / s
        icols.append(jnp.sum(ex / s, axis=0, keepdims=True))
        cand = jnp.where(l == m, iota_e, E)
        am = jnp.min(cand, axis=1, keepdims=True)
        oh = (am == iota_e).astype(F32)
        prior = jnp.dot(ltri.astype(jnp.bfloat16), oh.astype(jnp.bfloat16),
                        preferred_element_type=F32)
        cnt = cnt_ref[t:t + 1, :]
        posf = jnp.sum(oh * (prior + cnt), axis=1, keepdims=True)
        cnt_ref[t:t + 1, :] = cnt + jnp.sum(oh, axis=0, keepdims=True)
        keep = posf < CAP
        posc = jnp.minimum(posf, CAP - 1).astype(jnp.int32)
        dst = am * EC + t * CAP + posc
        dcols.append(jnp.where(keep, dst, am * EC + (EC - 1)))
        scols.append(jnp.where(keep, gate, 0.0))
    dst_ref[...] = jnp.concatenate(
        dcols + [jnp.zeros((BN, 128 - NT), jnp.int32)], axis=1)
    scale_ref[...] = jnp.concatenate(
        scols + [jnp.zeros((BN, 128 - NT), F32)], axis=1)
    impv = jnp.concatenate(icols, axis=1)
    imp_ref[...] = imp_ref[...] + jnp.broadcast_to(impv, imp_ref.shape)


def _bottom(x, fc1_w, fc1_b, fc2_w, fc2_b, gw2):
    return pl.pallas_call(
        _bottom_body,
        grid=(NB,),
        in_specs=[
            pl.BlockSpec((BN, DIN), lambda i: (i, 0)),
            pl.BlockSpec((DIN, D), lambda i: (0, 0)),
            pl.BlockSpec((1, D), lambda i: (0, 0)),
            pl.BlockSpec((D, D), lambda i: (0, 0)),
            pl.BlockSpec((1, D), lambda i: (0, 0)),
            pl.BlockSpec((D, NT * E), lambda i: (0, 0)),
        ],
        out_specs=[
            pl.BlockSpec((BN, D), lambda i: (i, 0)),
            pl.BlockSpec((BN, 128), lambda i: (i, 0)),
            pl.BlockSpec((BN, 128), lambda i: (i, 0)),
            pl.BlockSpec((8, NT * E), lambda i: (0, 0)),
        ],
        out_shape=[
            jax.ShapeDtypeStruct((N, D), F32),
            jax.ShapeDtypeStruct((N, 128), jnp.int32),
            jax.ShapeDtypeStruct((N, 128), F32),
            jax.ShapeDtypeStruct((8, NT * E), F32),
        ],
        scratch_shapes=[pltpu.VMEM((8, E), F32)],
    )(x, fc1_w, fc1_b.reshape(1, D), fc2_w, fc2_b.reshape(1, D), gw2)


def _sc_scatter(h, dstf):
    mesh = plsc.VectorSubcoreMesh(core_axis_name="c", subcore_axis_name="s")
    scw = N // NW

    @functools.partial(
        pl.kernel, mesh=mesh,
        out_type=jax.ShapeDtypeStruct((BUF_ROWS, D), F32),
        scratch_types=[
            pltpu.VMEM((scw,), jnp.int32),
            pltpu.VMEM((scw, D), F32),
        ],
    )
    def k(h_hbm, idx_hbm, buf_hbm, idx_v, rows_v):
        wid = lax.axis_index("s") * NC + lax.axis_index("c")
        base = wid * scw
        pltpu.sync_copy(h_hbm.at[pl.ds(base, scw)], rows_v)
        for t in range(NT):
            pltpu.sync_copy(idx_hbm.at[pl.ds(t * N + base, scw)], idx_v)
            pltpu.sync_copy(rows_v, buf_hbm.at[idx_v])

    return k(h, dstf)


def _ffn_body(buf_ref, w1_ref, b1_ref, w2_ref, b2_ref, out_ref):
    b = buf_ref[...]
    b = jnp.where(b != b, 0.0, b)
    b = jnp.clip(b, -1e30, 1e30)
    hid = jnp.dot(b.astype(jnp.bfloat16), w1_ref[0],
                  preferred_element_type=F32) + b1_ref[0]
    hid = jnp.maximum(hid, 0.0)
    out = jnp.dot(hid.astype(jnp.bfloat16), w2_ref[0],
                  preferred_element_type=F32) + b2_ref[0]
    out_ref[...] = out


def _ffn(buf, ew1, eb1, ew2, eb2):
    return pl.pallas_call(
        _ffn_body,
        grid=(E,),
        in_specs=[
            pl.BlockSpec((EC, D), lambda e: (e, 0)),
            pl.BlockSpec((1, D, DFF), lambda e: (e, 0, 0)),
            pl.BlockSpec((1, 1, DFF), lambda e: (e, 0, 0)),
            pl.BlockSpec((1, DFF, D), lambda e: (e, 0, 0)),
            pl.BlockSpec((1, 1, D), lambda e: (e, 0, 0)),
        ],
        out_specs=pl.BlockSpec((EC, D), lambda e: (e, 0)),
        out_shape=jax.ShapeDtypeStruct((BUF_ROWS, D), F32),
    )(buf, ew1, eb1, ew2, eb2)


def _sc_gather(src, dstf):
    mesh = plsc.VectorSubcoreMesh(core_axis_name="c", subcore_axis_name="s")
    gw = (NT * N) // NW
    gch = 128

    @functools.partial(
        pl.kernel, mesh=mesh,
        out_type=jax.ShapeDtypeStruct((NT * N, D), F32),
        scratch_types=[
            pltpu.VMEM((gch,), jnp.int32),
            pltpu.VMEM((gch, D), F32),
            pltpu.SemaphoreType.DMA,
        ],
    )
    def k(src_hbm, idx_hbm, y_hbm, idx_v, rows_v, sem):
        wid = lax.axis_index("s") * NC + lax.axis_index("c")
        for ci in range(gw // gch):
            base = wid * gw + ci * gch
            pltpu.sync_copy(idx_hbm.at[pl.ds(base, gch)], idx_v)
            pltpu.async_copy(src_hbm.at[idx_v], rows_v, sem).wait()
            pltpu.sync_copy(rows_v, y_hbm.at[pl.ds(base, gch)])

    return k(src, dstf)


def _tower_body(y_ref, scale_ref, w1_ref, b1_ref, w2_ref, b2_ref, imp_ref,
                out_ref, aux_ref):
    i = pl.program_id(0)
    t = pl.program_id(1)

    @pl.when((i == 0) & (t == 0))
    def _():
        auxv = 0.0
        for tt in range(NT):
            imp = imp_ref[0:1, tt * E:(tt + 1) * E]
            mean = jnp.sum(imp) / E
            var = jnp.sum((imp - mean) ** 2) / E
            auxv = auxv + var / (mean * mean + 1e-9)
        aux_ref[...] = jnp.full((8, 128), auxv / NT, F32)

    lane = lax.broadcasted_iota(jnp.int32, (1, 128), 1)
    sb = scale_ref[...]
    s = jnp.sum(sb * (lane == t).astype(F32), axis=1, keepdims=True)
    z = y_ref[...] * s
    hid = jnp.maximum(
        jnp.dot(z.astype(jnp.bfloat16), w1_ref[0], preferred_element_type=F32)
        + b1_ref[0], 0.0)
    w2r = w2_ref[0]
    b2s = jnp.max(b2_ref[0], axis=1, keepdims=True)
    t2 = jnp.sum(hid * w2r, axis=1, keepdims=True) + b2s

    @pl.when(t == 0)
    def _():
        out_ref[...] = jnp.zeros((BT, 128), F32)

    out_ref[...] = jnp.where(lane == t, t2, out_ref[...])


def _towers(y, scaleq, tw1, tb1, tw2r, tb2b, imp):
    return pl.pallas_call(
        _tower_body,
        grid=(NBT, NT),
        in_specs=[
            pl.BlockSpec((BT, D), lambda i, t: (t * NBT + i, 0)),
            pl.BlockSpec((BT, 128), lambda i, t: (i, 0)),
            pl.BlockSpec((1, D, TH), lambda i, t: (t, 0, 0)),
            pl.BlockSpec((1, 1, TH), lambda i, t: (t, 0, 0)),
            pl.BlockSpec((1, 1, TH), lambda i, t: (t, 0, 0)),
            pl.BlockSpec((1, 1, TH), lambda i, t: (t, 0, 0)),
            pl.BlockSpec((8, NT * E), lambda i, t: (0, 0)),
        ],
        out_specs=[
            pl.BlockSpec((BT, 128), lambda i, t: (i, 0)),
            pl.BlockSpec((8, 128), lambda i, t: (0, 0)),
        ],
        out_shape=[
            jax.ShapeDtypeStruct((N, 128), F32),
            jax.ShapeDtypeStruct((8, 128), F32),
        ],
    )(y, scaleq, tw1, tb1, tw2r, tb2b, imp)


def kernel(x, fc1_w, fc1_b, fc2_w, fc2_b, gate_w, expert_w1, expert_b1,
           expert_w2, expert_b2, tower_w1, tower_b1, tower_w2, tower_b2):
    gw2 = gate_w.transpose(1, 0, 2).reshape(D, NT * E)
    h, dstq, scaleq, imp = _bottom(x, fc1_w, fc1_b, fc2_w, fc2_b, gw2)
    dstf = dstq[:, :NT].T.reshape(NT * N)
    buf = _sc_scatter(h, dstf)
    outb = _ffn(buf,
                expert_w1.astype(jnp.bfloat16),
                expert_b1.reshape(E, 1, DFF),
                expert_w2.astype(jnp.bfloat16),
                expert_b2.reshape(E, 1, D))
    y = _sc_gather(outb, dstf)
    tl, auxm = _towers(y, scaleq,
                       tower_w1.astype(jnp.bfloat16),
                       tower_b1.reshape(NT, 1, TH),
                       tower_w2.reshape(NT, 1, TH),
                       jnp.broadcast_to(tower_b2.reshape(NT, 1, 1), (NT, 1, TH)),
                       imp)
    logits = tl[:, :NT].T
    return logits, auxm[0, 0]
